# Initial kernel scaffold; baseline (speedup 1.0000x reference)
import jax
import jax.numpy as jnp
from jax.experimental import pallas as pl


def kernel(x, params, edge_index, batch):
    raise NotImplementedError("write your pallas kernel here")


# speedup vs baseline: 1.0363x; 1.0363x over previous
import functools

import jax
import jax.numpy as jnp
import numpy as np
from jax.experimental import pallas as pl
from jax.experimental.pallas import tpu as pltpu

N = 10000
E = 320000
IN_CH = 128
HEADS = 8
OUT_CH = 64
HID = HEADS * OUT_CH
NUM_GRAPHS = 64

ROW_BLK = 1000


def _linear_body(x_ref, w_ref, b_ref, o_ref):
    o_ref[...] = (
        jnp.dot(x_ref[...], w_ref[...], preferred_element_type=jnp.float32)
        + b_ref[...]
    )


def _linear(x, w, b):
    n, fin = x.shape
    fout = w.shape[1]
    blk = ROW_BLK if n % ROW_BLK == 0 else n
    grid = n // blk
    return pl.pallas_call(
        _linear_body,
        grid=(grid,),
        in_specs=[
            pl.BlockSpec((blk, fin), lambda i: (i, 0)),
            pl.BlockSpec((fin, fout), lambda i: (0, 0)),
            pl.BlockSpec((1, fout), lambda i: (0, 0)),
        ],
        out_specs=pl.BlockSpec((blk, fout), lambda i: (i, 0)),
        out_shape=jax.ShapeDtypeStruct((n, fout), jnp.float32),
    )(x, w, b.reshape(1, fout))


def _epilogue_body(agg_ref, den_ref, skip_ref, res_ref, g_ref, b_ref, o_ref):
    den = den_ref[...]
    den = jnp.repeat(den, OUT_CH, axis=1)
    h = agg_ref[...] / (den + 1e-16) + skip_ref[...]
    h = jnp.maximum(h, 0.0) + res_ref[...]
    mu = jnp.mean(h, axis=1, keepdims=True)
    var = jnp.mean((h - mu) ** 2, axis=1, keepdims=True)
    o_ref[...] = (h - mu) / jnp.sqrt(var + 1e-5) * g_ref[...] + b_ref[...]


def _epilogue(agg, denom, skip, res, g, b):
    blk = ROW_BLK
    grid = N // blk
    return pl.pallas_call(
        _epilogue_body,
        grid=(grid,),
        in_specs=[
            pl.BlockSpec((blk, HID), lambda i: (i, 0)),
            pl.BlockSpec((blk, HEADS), lambda i: (i, 0)),
            pl.BlockSpec((blk, HID), lambda i: (i, 0)),
            pl.BlockSpec((blk, HID), lambda i: (i, 0)),
            pl.BlockSpec((1, HID), lambda i: (0, 0)),
            pl.BlockSpec((1, HID), lambda i: (0, 0)),
        ],
        out_specs=pl.BlockSpec((blk, HID), lambda i: (i, 0)),
        out_shape=jax.ShapeDtypeStruct((N, HID), jnp.float32),
    )(agg, denom, skip, res, g.reshape(1, HID), b.reshape(1, HID))


def _edge_phase(q, k, v, src, dst):
    alpha = jnp.sum(q[dst] * k[src], axis=-1) / np.sqrt(OUT_CH)
    m = jnp.max(alpha, axis=0)
    ex = jnp.exp(alpha - m[None, :])
    denom = jax.ops.segment_sum(ex, dst, num_segments=N)
    msg = v[src] * ex[..., None]
    agg = jax.ops.segment_sum(msg, dst, num_segments=N).reshape(N, HID)
    return agg, denom


def _head_body(g_ref, w1_ref, b1_ref, w2_ref, b2_ref, o_ref):
    h = jnp.dot(g_ref[...], w1_ref[...], preferred_element_type=jnp.float32)
    h = jnp.maximum(h + b1_ref[...], 0.0)
    o_ref[...] = jnp.dot(h, w2_ref[...], preferred_element_type=jnp.float32) + b2_ref[...]


def _head(graph, hp):
    return pl.pallas_call(
        _head_body,
        grid=(1,),
        in_specs=[
            pl.BlockSpec((NUM_GRAPHS, HID), lambda i: (0, 0)),
            pl.BlockSpec((HID, OUT_CH), lambda i: (0, 0)),
            pl.BlockSpec((1, OUT_CH), lambda i: (0, 0)),
            pl.BlockSpec((OUT_CH, 1), lambda i: (0, 0)),
            pl.BlockSpec((1, 1), lambda i: (0, 0)),
        ],
        out_specs=pl.BlockSpec((NUM_GRAPHS, 1), lambda i: (0, 0)),
        out_shape=jax.ShapeDtypeStruct((NUM_GRAPHS, 1), jnp.float32),
    )(graph, hp["W1"], hp["b1"].reshape(1, OUT_CH), hp["W2"],
      hp["b2"].reshape(1, 1))


def kernel(x, params, edge_index, batch):
    src = edge_index[0]
    dst = edge_index[1]
    cs = params["convs"]
    h = x
    res = jnp.zeros((N, HID), jnp.float32)
    for l in range(4):
        p = cs[l]
        wall = jnp.concatenate([p["Wq"], p["Wk"], p["Wv"], p["Ws"]], axis=1)
        ball = jnp.concatenate([p["bq"], p["bk"], p["bv"], p["bs"]], axis=0)
        qkvs = _linear(h, wall, ball)
        q = qkvs[:, 0 * HID:1 * HID].reshape(N, HEADS, OUT_CH)
        k = qkvs[:, 1 * HID:2 * HID].reshape(N, HEADS, OUT_CH)
        v = qkvs[:, 2 * HID:3 * HID].reshape(N, HEADS, OUT_CH)
        skip = qkvs[:, 3 * HID:4 * HID]
        agg, denom = _edge_phase(q, k, v, src, dst)
        h = _epilogue(agg, denom, skip, res, p["ln_g"], p["ln_b"])
        res = h
    sums = jax.ops.segment_sum(h, batch, num_segments=NUM_GRAPHS)
    cnt = jax.ops.segment_sum(jnp.ones((N,), jnp.float32), batch,
                              num_segments=NUM_GRAPHS)
    graph = sums / jnp.maximum(cnt, 1.0)[:, None]
    return _head(graph, params["head"])


# speedup vs baseline: 10.5367x; 10.1675x over previous
import functools

import jax
import jax.numpy as jnp
from jax import lax
from jax.experimental import pallas as pl
from jax.experimental.pallas import tpu as pltpu
from jax.experimental.pallas import tpu_sc as plsc

N = 10000
E = 320000
IN_CH = 128
HEADS = 8
OUT_CH = 64
HID = HEADS * OUT_CH
NUM_GRAPHS = 64

ROW_BLK = 1000
EDGE_BLK = 2000
NC = 2
NS = 16
NW = NC * NS
EPW = E // NW
G = 80
NCHUNK = EPW // G


def _proj_body(x_ref, w_ref, b_ref, q_ref, k_ref, v_ref, s_ref):
    y = (jnp.dot(x_ref[...], w_ref[...], preferred_element_type=jnp.float32)
         + b_ref[...])
    q_ref[...] = y[:, 0 * HID:1 * HID]
    k_ref[...] = y[:, 1 * HID:2 * HID]
    v_ref[...] = y[:, 2 * HID:3 * HID]
    s_ref[...] = y[:, 3 * HID:4 * HID]


def _proj(x, w, b):
    n, fin = x.shape
    blk = ROW_BLK
    out = jax.ShapeDtypeStruct((n, HID), jnp.float32)
    return pl.pallas_call(
        _proj_body,
        grid=(n // blk,),
        in_specs=[
            pl.BlockSpec((blk, fin), lambda i: (i, 0)),
            pl.BlockSpec((fin, 4 * HID), lambda i: (0, 0)),
            pl.BlockSpec((1, 4 * HID), lambda i: (0, 0)),
        ],
        out_specs=[pl.BlockSpec((blk, HID), lambda i: (i, 0))] * 4,
        out_shape=[out, out, out, out],
    )(x, w, b.reshape(1, 4 * HID))


_sc_mesh = plsc.VectorSubcoreMesh(core_axis_name="c", subcore_axis_name="s")


@functools.partial(
    pl.kernel,
    mesh=_sc_mesh,
    out_type=[jax.ShapeDtypeStruct((E, HID), jnp.float32)] * 3,
    scratch_types=[
        pltpu.VMEM((G,), jnp.int32),
        pltpu.VMEM((G, HID), jnp.float32),
        pltpu.SemaphoreType.DMA,
    ],
)
def _sc_gather(q_hbm, k_hbm, v_hbm, src_hbm, dst_hbm,
               qd_hbm, ks_hbm, vs_hbm, idx_v, rows_v, sem):
    wid = lax.axis_index("s") * NC + lax.axis_index("c")
    base = wid * EPW

    def one_table(tab, idxarr, out):
        def body(i, carry):
            off = base + i * G
            pltpu.sync_copy(idxarr.at[pl.ds(off, G)], idx_v)
            pltpu.async_copy(tab.at[idx_v], rows_v, sem).wait()
            pltpu.sync_copy(rows_v, out.at[pl.ds(off, G)])
            return carry
        lax.fori_loop(0, NCHUNK, body, 0)

    one_table(q_hbm, dst_hbm, qd_hbm)
    one_table(k_hbm, src_hbm, ks_hbm)
    one_table(v_hbm, src_hbm, vs_hbm)


def _edge_math_body(qd_ref, ks_ref, vs_ref,
                    m0_ref, m1_ref, m2_ref, m3_ref, m4_ref):
    blk = qd_ref.shape[0]
    prod = qd_ref[...] * ks_ref[...]
    alpha = jnp.sum(prod.reshape(blk, HEADS, OUT_CH), axis=-1) * 0.125
    ex = jnp.exp(alpha)
    exfull = jnp.repeat(ex, OUT_CH, axis=1)
    m = vs_ref[...] * exfull
    m0_ref[...] = m[:, 0:128]
    m1_ref[...] = m[:, 128:256]
    m2_ref[...] = m[:, 256:384]
    m3_ref[...] = m[:, 384:512]
    m4_ref[...] = jnp.concatenate(
        [ex, jnp.zeros((blk, 120), jnp.float32)], axis=1)


def _edge_math(qd, ks, vs):
    blk = EDGE_BLK
    mout = jax.ShapeDtypeStruct((E, 128), jnp.float32)
    return pl.pallas_call(
        _edge_math_body,
        grid=(E // blk,),
        in_specs=[pl.BlockSpec((blk, HID), lambda i: (i, 0))] * 3,
        out_specs=[pl.BlockSpec((blk, 128), lambda i: (i, 0))] * 5,
        out_shape=[mout, mout, mout, mout, mout],
    )(qd, ks, vs)


@functools.partial(
    pl.kernel,
    mesh=_sc_mesh,
    out_type=[jax.ShapeDtypeStruct((5, N, 128), jnp.float32),
              jax.ShapeDtypeStruct((5, N, 128), jnp.float32)],
    scratch_types=[
        pltpu.VMEM((G,), jnp.int32),
        pltpu.VMEM((G, 128), jnp.float32),
        pltpu.VMEM((G, 128), jnp.float32),
        pltpu.VMEM_SHARED((N, 128), jnp.float32),
        pltpu.SemaphoreType.DMA,
    ],
)
def _sc_scatter(m0_hbm, m1_hbm, m2_hbm, m3_hbm, m4_hbm, dst_hbm, z128_hbm,
                agg0_hbm, agg1_hbm, idx_v, mbuf, zvb, acc, sem):
    cid = lax.axis_index("c")
    sid = lax.axis_index("s")
    wid = sid * NC + cid
    base = wid * EPW

    pltpu.sync_copy(z128_hbm.at[pl.ds(0, G)], zvb)

    rstart = sid * 640
    rend = jnp.minimum(rstart + 640, N)

    def rowchunks(fn):
        for j in range(8):
            off = rstart + j * G
            @pl.when(off < rend)
            def _():
                fn(pl.ds(off, G))

    for g, mg in enumerate((m0_hbm, m1_hbm, m2_hbm, m3_hbm, m4_hbm)):
        rowchunks(lambda r: pltpu.sync_copy(zvb, acc.at[r]))
        plsc.subcore_barrier()

        def body(i, carry):
            off = base + i * G
            pltpu.sync_copy(dst_hbm.at[pl.ds(off, G)], idx_v)
            pltpu.sync_copy(mg.at[pl.ds(off, G)], mbuf)
            pltpu.sync_copy(mbuf, acc.at[idx_v], add=True)
            return carry
        lax.fori_loop(0, NCHUNK, body, 0)
        plsc.subcore_barrier()

        def flush(out):
            def one(r):
                pltpu.sync_copy(acc.at[r], mbuf)
                pltpu.sync_copy(mbuf, out.at[g, r])
            rowchunks(one)

        @pl.when(cid == 0)
        def _():
            flush(agg0_hbm)

        @pl.when(cid == 1)
        def _():
            flush(agg1_hbm)

        plsc.subcore_barrier()


def _epilogue_body(a00, a01, a02, a03, a04, a10, a11, a12, a13, a14,
                   skip_ref, res_ref, g_ref, b_ref, o_ref):
    agg = jnp.concatenate(
        [a00[...] + a10[...], a01[...] + a11[...],
         a02[...] + a12[...], a03[...] + a13[...]], axis=1)
    den8 = (a04[...] + a14[...])[:, 0:8]
    den_full = jnp.repeat(den8, OUT_CH, axis=1)
    h = agg / (den_full + 1e-16) + skip_ref[...]
    h = jnp.maximum(h, 0.0) + res_ref[...]
    mu = jnp.mean(h, axis=1, keepdims=True)
    var = jnp.mean((h - mu) ** 2, axis=1, keepdims=True)
    o_ref[...] = (h - mu) / jnp.sqrt(var + 1e-5) * g_ref[...] + b_ref[...]


def _epilogue(agg0, agg1, skip, res, g, b):
    blk = ROW_BLK
    aspec = [pl.BlockSpec((blk, 128), lambda i: (i, 0))] * 10
    return pl.pallas_call(
        _epilogue_body,
        grid=(N // blk,),
        in_specs=aspec + [
            pl.BlockSpec((blk, HID), lambda i: (i, 0)),
            pl.BlockSpec((blk, HID), lambda i: (i, 0)),
            pl.BlockSpec((1, HID), lambda i: (0, 0)),
            pl.BlockSpec((1, HID), lambda i: (0, 0)),
        ],
        out_specs=pl.BlockSpec((blk, HID), lambda i: (i, 0)),
        out_shape=jax.ShapeDtypeStruct((N, HID), jnp.float32),
    )(agg0[0], agg0[1], agg0[2], agg0[3], agg0[4],
      agg1[0], agg1[1], agg1[2], agg1[3], agg1[4],
      skip, res, g.reshape(1, HID), b.reshape(1, HID))


def _pool_body(h_ref, b_ref, sums_ref, cnt_ref):
    blk = h_ref.shape[0]
    oh = (b_ref[...] == lax.broadcasted_iota(jnp.int32, (1, NUM_GRAPHS), 1)
          ).astype(jnp.float32)
    part = lax.dot_general(oh, h_ref[...], (((0,), (0,)), ((), ())),
                           preferred_element_type=jnp.float32)
    cpart = lax.dot_general(oh, jnp.ones((blk, 128), jnp.float32),
                            (((0,), (0,)), ((), ())),
                            preferred_element_type=jnp.float32)

    @pl.when(pl.program_id(0) == 0)
    def _():
        sums_ref[...] = jnp.zeros_like(sums_ref)
        cnt_ref[...] = jnp.zeros_like(cnt_ref)

    sums_ref[...] += part
    cnt_ref[...] += cpart


def _pool(h, batch2):
    blk = ROW_BLK
    return pl.pallas_call(
        _pool_body,
        grid=(N // blk,),
        in_specs=[
            pl.BlockSpec((blk, HID), lambda i: (i, 0)),
            pl.BlockSpec((blk, 1), lambda i: (i, 0)),
        ],
        out_specs=[pl.BlockSpec((NUM_GRAPHS, HID), lambda i: (0, 0)),
                   pl.BlockSpec((NUM_GRAPHS, 128), lambda i: (0, 0))],
        out_shape=[jax.ShapeDtypeStruct((NUM_GRAPHS, HID), jnp.float32),
                   jax.ShapeDtypeStruct((NUM_GRAPHS, 128), jnp.float32)],
    )(h, batch2)


def _head_body(s_ref, c_ref, w1_ref, b1_ref, w2_ref, b2_ref, o_ref):
    cnt = jnp.maximum(c_ref[...], 1.0)
    graph = (s_ref[...].reshape(NUM_GRAPHS, 4, 128) / cnt[:, None, :]
             ).reshape(NUM_GRAPHS, HID)
    h = jnp.dot(graph, w1_ref[...], preferred_element_type=jnp.float32)
    h = jnp.maximum(h + b1_ref[...], 0.0)
    o_ref[...] = (jnp.dot(h, w2_ref[...], preferred_element_type=jnp.float32)
                  + b2_ref[...])


def _head(sums, cnt, hp):
    return pl.pallas_call(
        _head_body,
        grid=(1,),
        in_specs=[
            pl.BlockSpec((NUM_GRAPHS, HID), lambda i: (0, 0)),
            pl.BlockSpec((NUM_GRAPHS, 128), lambda i: (0, 0)),
            pl.BlockSpec((HID, OUT_CH), lambda i: (0, 0)),
            pl.BlockSpec((1, OUT_CH), lambda i: (0, 0)),
            pl.BlockSpec((OUT_CH, 1), lambda i: (0, 0)),
            pl.BlockSpec((1, 1), lambda i: (0, 0)),
        ],
        out_specs=pl.BlockSpec((NUM_GRAPHS, 1), lambda i: (0, 0)),
        out_shape=jax.ShapeDtypeStruct((NUM_GRAPHS, 1), jnp.float32),
    )(sums, cnt, hp["W1"], hp["b1"].reshape(1, OUT_CH), hp["W2"],
      hp["b2"].reshape(1, 1))


def kernel(x, params, edge_index, batch):
    src = edge_index[0]
    dst = edge_index[1]
    z128 = jnp.zeros((G, 128), jnp.float32)
    cs = params["convs"]
    h = x
    res = jnp.zeros((N, HID), jnp.float32)
    for l in range(4):
        p = cs[l]
        wall = jnp.concatenate([p["Wq"], p["Wk"], p["Wv"], p["Ws"]], axis=1)
        ball = jnp.concatenate([p["bq"], p["bk"], p["bv"], p["bs"]], axis=0)
        q, k, v, skip = _proj(h, wall, ball)
        qd, ks, vs = _sc_gather(q, k, v, src, dst)
        m0, m1, m2, m3, m4 = _edge_math(qd, ks, vs)
        agg0, agg1 = _sc_scatter(m0, m1, m2, m3, m4, dst, z128)
        h = _epilogue(agg0, agg1, skip, res, p["ln_g"], p["ln_b"])
        res = h
    sums, cnt = _pool(h, batch.reshape(N, 1))
    return _head(sums, cnt, params["head"])


# speedup vs baseline: 14.0202x; 1.3306x over previous
import functools

import jax
import jax.numpy as jnp
from jax import lax
from jax.experimental import pallas as pl
from jax.experimental.pallas import tpu as pltpu
from jax.experimental.pallas import tpu_sc as plsc

N = 10000
E = 320000
IN_CH = 128
HEADS = 8
OUT_CH = 64
HID = HEADS * OUT_CH
NUM_GRAPHS = 64

ROW_BLK = 1000
EDGE_BLK = 2000
NC = 2
NS = 16
NW = NC * NS
EPW = E // NW
G = 80
NCHUNK = EPW // G


def _proj_body(x_ref, w_ref, b_ref, q_ref, k_ref, v_ref, s_ref):
    y = (jnp.dot(x_ref[...], w_ref[...], preferred_element_type=jnp.float32)
         + b_ref[...])
    q_ref[...] = y[:, 0 * HID:1 * HID]
    k_ref[...] = y[:, 1 * HID:2 * HID]
    v_ref[...] = y[:, 2 * HID:3 * HID]
    s_ref[...] = y[:, 3 * HID:4 * HID]


def _proj(x, w, b):
    n, fin = x.shape
    blk = ROW_BLK
    out = jax.ShapeDtypeStruct((n, HID), jnp.float32)
    return pl.pallas_call(
        _proj_body,
        grid=(n // blk,),
        in_specs=[
            pl.BlockSpec((blk, fin), lambda i: (i, 0)),
            pl.BlockSpec((fin, 4 * HID), lambda i: (0, 0)),
            pl.BlockSpec((1, 4 * HID), lambda i: (0, 0)),
        ],
        out_specs=[pl.BlockSpec((blk, HID), lambda i: (i, 0))] * 4,
        out_shape=[out, out, out, out],
    )(x, w, b.reshape(1, 4 * HID))


_sc_mesh = plsc.VectorSubcoreMesh(core_axis_name="c", subcore_axis_name="s")


@functools.partial(
    pl.kernel,
    mesh=_sc_mesh,
    out_type=[jax.ShapeDtypeStruct((E, HID), jnp.float32)] * 3,
    scratch_types=[
        pltpu.VMEM((G,), jnp.int32),
        pltpu.VMEM((G,), jnp.int32),
        pltpu.VMEM((G, HID), jnp.float32),
        pltpu.VMEM((G, HID), jnp.float32),
        pltpu.SemaphoreType.DMA,
        pltpu.SemaphoreType.DMA,
    ],
)
def _sc_gather(q_hbm, k_hbm, v_hbm, src_hbm, dst_hbm,
               qd_hbm, ks_hbm, vs_hbm, idx0, idx1, rows0, rows1,
               sem0, sem1):
    wid = lax.axis_index("s") * NC + lax.axis_index("c")
    base = wid * EPW
    idxb = (idx0, idx1)
    rowsb = (rows0, rows1)
    semb = (sem0, sem1)

    def one_table(tab, idxarr, out):
        def start(i, b):
            pltpu.sync_copy(idxarr.at[pl.ds(base + i * G, G)], idxb[b])
            pltpu.async_copy(tab.at[idxb[b]], rowsb[b], semb[b])

        def drain(i, b):
            pltpu.make_async_copy(tab.at[idxb[b]], rowsb[b], semb[b]).wait()
            pltpu.sync_copy(rowsb[b], out.at[pl.ds(base + i * G, G)])

        start(0, 0)

        def pair(j, c):
            i1 = 2 * j + 1
            start(i1, 1)
            drain(i1 - 1, 0)
            start(i1 + 1, 0)
            drain(i1, 1)
            return c
        lax.fori_loop(0, (NCHUNK - 1) // 2, pair, 0)
        drain(NCHUNK - 1, (NCHUNK - 1) % 2)

    one_table(q_hbm, dst_hbm, qd_hbm)
    one_table(k_hbm, src_hbm, ks_hbm)
    one_table(v_hbm, src_hbm, vs_hbm)


def _edge_math_body(qd_ref, ks_ref, vs_ref,
                    m0_ref, m1_ref, m2_ref, m3_ref, m4_ref):
    blk = qd_ref.shape[0]
    prod = qd_ref[...] * ks_ref[...]
    alpha = jnp.sum(prod.reshape(blk, HEADS, OUT_CH), axis=-1) * 0.125
    ex = jnp.exp(alpha)
    exfull = jnp.repeat(ex, OUT_CH, axis=1)
    m = vs_ref[...] * exfull
    m0_ref[...] = m[:, 0:128]
    m1_ref[...] = m[:, 128:256]
    m2_ref[...] = m[:, 256:384]
    m3_ref[...] = m[:, 384:512]
    m4_ref[...] = jnp.concatenate(
        [ex, jnp.zeros((blk, 120), jnp.float32)], axis=1)


def _edge_math(qd, ks, vs):
    blk = EDGE_BLK
    mout = jax.ShapeDtypeStruct((E, 128), jnp.float32)
    return pl.pallas_call(
        _edge_math_body,
        grid=(E // blk,),
        in_specs=[pl.BlockSpec((blk, HID), lambda i: (i, 0))] * 3,
        out_specs=[pl.BlockSpec((blk, 128), lambda i: (i, 0))] * 5,
        out_shape=[mout, mout, mout, mout, mout],
    )(qd, ks, vs)


@functools.partial(
    pl.kernel,
    mesh=_sc_mesh,
    out_type=[jax.ShapeDtypeStruct((5, N, 128), jnp.float32),
              jax.ShapeDtypeStruct((5, N, 128), jnp.float32)],
    scratch_types=[
        pltpu.VMEM((G,), jnp.int32),
        pltpu.VMEM((G,), jnp.int32),
        pltpu.VMEM((G, 128), jnp.float32),
        pltpu.VMEM((G, 128), jnp.float32),
        pltpu.VMEM((G, 128), jnp.float32),
        pltpu.VMEM_SHARED((N, 128), jnp.float32),
        pltpu.SemaphoreType.DMA,
        pltpu.SemaphoreType.DMA,
    ],
)
def _sc_scatter(m0_hbm, m1_hbm, m2_hbm, m3_hbm, m4_hbm, dst_hbm, z128_hbm,
                agg0_hbm, agg1_hbm, idxa, idxb, mbufa, mbufb, zvb, acc,
                sema, semb):
    cid = lax.axis_index("c")
    sid = lax.axis_index("s")
    wid = sid * NC + cid
    base = wid * EPW

    pltpu.sync_copy(z128_hbm.at[pl.ds(0, G)], zvb)

    rstart = sid * 640
    rend = jnp.minimum(rstart + 640, N)

    def rowchunks(fn):
        for j in range(8):
            off = rstart + j * G
            @pl.when(off < rend)
            def _():
                fn(pl.ds(off, G))

    idxs = (idxa, idxb)
    mbufs = (mbufa, mbufb)
    sems = (sema, semb)

    for g, mg in enumerate((m0_hbm, m1_hbm, m2_hbm, m3_hbm, m4_hbm)):
        rowchunks(lambda r: pltpu.sync_copy(zvb, acc.at[r]))
        plsc.subcore_barrier()

        def start(i, b):
            pltpu.sync_copy(dst_hbm.at[pl.ds(base + i * G, G)], idxs[b])
            pltpu.async_copy(mg.at[pl.ds(base + i * G, G)], mbufs[b],
                             sems[b])

        def commit(i, b):
            pltpu.make_async_copy(mg.at[pl.ds(base + i * G, G)], mbufs[b],
                                  sems[b]).wait()
            pltpu.sync_copy(mbufs[b], acc.at[idxs[b]], add=True)

        start(0, 0)

        def pair(j, c):
            i1 = 2 * j + 1
            start(i1, 1)
            commit(i1 - 1, 0)
            start(i1 + 1, 0)
            commit(i1, 1)
            return c
        lax.fori_loop(0, (NCHUNK - 1) // 2, pair, 0)
        commit(NCHUNK - 1, (NCHUNK - 1) % 2)
        plsc.subcore_barrier()

        def flush(out):
            def one(r):
                pltpu.sync_copy(acc.at[r], mbufa)
                pltpu.sync_copy(mbufa, out.at[g, r])
            rowchunks(one)

        @pl.when(cid == 0)
        def _():
            flush(agg0_hbm)

        @pl.when(cid == 1)
        def _():
            flush(agg1_hbm)

        plsc.subcore_barrier()


def _epilogue_body(a00, a01, a02, a03, a04, a10, a11, a12, a13, a14,
                   skip_ref, res_ref, g_ref, b_ref, o_ref):
    agg = jnp.concatenate(
        [a00[...] + a10[...], a01[...] + a11[...],
         a02[...] + a12[...], a03[...] + a13[...]], axis=1)
    den8 = (a04[...] + a14[...])[:, 0:8]
    den_full = jnp.repeat(den8, OUT_CH, axis=1)
    h = agg / (den_full + 1e-16) + skip_ref[...]
    h = jnp.maximum(h, 0.0) + res_ref[...]
    mu = jnp.mean(h, axis=1, keepdims=True)
    var = jnp.mean((h - mu) ** 2, axis=1, keepdims=True)
    o_ref[...] = (h - mu) / jnp.sqrt(var + 1e-5) * g_ref[...] + b_ref[...]


def _epilogue(agg0, agg1, skip, res, g, b):
    blk = ROW_BLK
    aspec = [pl.BlockSpec((blk, 128), lambda i: (i, 0))] * 10
    return pl.pallas_call(
        _epilogue_body,
        grid=(N // blk,),
        in_specs=aspec + [
            pl.BlockSpec((blk, HID), lambda i: (i, 0)),
            pl.BlockSpec((blk, HID), lambda i: (i, 0)),
            pl.BlockSpec((1, HID), lambda i: (0, 0)),
            pl.BlockSpec((1, HID), lambda i: (0, 0)),
        ],
        out_specs=pl.BlockSpec((blk, HID), lambda i: (i, 0)),
        out_shape=jax.ShapeDtypeStruct((N, HID), jnp.float32),
    )(agg0[0], agg0[1], agg0[2], agg0[3], agg0[4],
      agg1[0], agg1[1], agg1[2], agg1[3], agg1[4],
      skip, res, g.reshape(1, HID), b.reshape(1, HID))


def _pool_body(h_ref, b_ref, sums_ref, cnt_ref):
    blk = h_ref.shape[0]
    oh = (b_ref[...] == lax.broadcasted_iota(jnp.int32, (1, NUM_GRAPHS), 1)
          ).astype(jnp.float32)
    part = lax.dot_general(oh, h_ref[...], (((0,), (0,)), ((), ())),
                           preferred_element_type=jnp.float32)
    cpart = lax.dot_general(oh, jnp.ones((blk, 128), jnp.float32),
                            (((0,), (0,)), ((), ())),
                            preferred_element_type=jnp.float32)

    @pl.when(pl.program_id(0) == 0)
    def _():
        sums_ref[...] = jnp.zeros_like(sums_ref)
        cnt_ref[...] = jnp.zeros_like(cnt_ref)

    sums_ref[...] += part
    cnt_ref[...] += cpart


def _pool(h, batch2):
    blk = ROW_BLK
    return pl.pallas_call(
        _pool_body,
        grid=(N // blk,),
        in_specs=[
            pl.BlockSpec((blk, HID), lambda i: (i, 0)),
            pl.BlockSpec((blk, 1), lambda i: (i, 0)),
        ],
        out_specs=[pl.BlockSpec((NUM_GRAPHS, HID), lambda i: (0, 0)),
                   pl.BlockSpec((NUM_GRAPHS, 128), lambda i: (0, 0))],
        out_shape=[jax.ShapeDtypeStruct((NUM_GRAPHS, HID), jnp.float32),
                   jax.ShapeDtypeStruct((NUM_GRAPHS, 128), jnp.float32)],
    )(h, batch2)


def _head_body(s_ref, c_ref, w1_ref, b1_ref, w2_ref, b2_ref, o_ref):
    cnt = jnp.maximum(c_ref[...], 1.0)
    graph = (s_ref[...].reshape(NUM_GRAPHS, 4, 128) / cnt[:, None, :]
             ).reshape(NUM_GRAPHS, HID)
    h = jnp.dot(graph, w1_ref[...], preferred_element_type=jnp.float32)
    h = jnp.maximum(h + b1_ref[...], 0.0)
    o_ref[...] = (jnp.dot(h, w2_ref[...], preferred_element_type=jnp.float32)
                  + b2_ref[...])


def _head(sums, cnt, hp):
    return pl.pallas_call(
        _head_body,
        grid=(1,),
        in_specs=[
            pl.BlockSpec((NUM_GRAPHS, HID), lambda i: (0, 0)),
            pl.BlockSpec((NUM_GRAPHS, 128), lambda i: (0, 0)),
            pl.BlockSpec((HID, OUT_CH), lambda i: (0, 0)),
            pl.BlockSpec((1, OUT_CH), lambda i: (0, 0)),
            pl.BlockSpec((OUT_CH, 1), lambda i: (0, 0)),
            pl.BlockSpec((1, 1), lambda i: (0, 0)),
        ],
        out_specs=pl.BlockSpec((NUM_GRAPHS, 1), lambda i: (0, 0)),
        out_shape=jax.ShapeDtypeStruct((NUM_GRAPHS, 1), jnp.float32),
    )(sums, cnt, hp["W1"], hp["b1"].reshape(1, OUT_CH), hp["W2"],
      hp["b2"].reshape(1, 1))


def kernel(x, params, edge_index, batch):
    src = edge_index[0]
    dst = edge_index[1]
    z128 = jnp.zeros((G, 128), jnp.float32)
    cs = params["convs"]
    h = x
    res = jnp.zeros((N, HID), jnp.float32)
    for l in range(4):
        p = cs[l]
        wall = jnp.concatenate([p["Wq"], p["Wk"], p["Wv"], p["Ws"]], axis=1)
        ball = jnp.concatenate([p["bq"], p["bk"], p["bv"], p["bs"]], axis=0)
        q, k, v, skip = _proj(h, wall, ball)
        qd, ks, vs = _sc_gather(q, k, v, src, dst)
        m0, m1, m2, m3, m4 = _edge_math(qd, ks, vs)
        agg0, agg1 = _sc_scatter(m0, m1, m2, m3, m4, dst, z128)
        h = _epilogue(agg0, agg1, skip, res, p["ln_g"], p["ln_b"])
        res = h
    sums, cnt = _pool(h, batch.reshape(N, 1))
    return _head(sums, cnt, params["head"])


# speedup vs baseline: 17.7536x; 1.2663x over previous
import functools

import jax
import jax.numpy as jnp
from jax import lax
from jax.experimental import pallas as pl
from jax.experimental.pallas import tpu as pltpu
from jax.experimental.pallas import tpu_sc as plsc

N = 10000
E = 320000
IN_CH = 128
HEADS = 8
OUT_CH = 64
HID = HEADS * OUT_CH
NUM_GRAPHS = 64

ROW_BLK = 1000
EDGE_BLK = 2000
NC = 2
NS = 16
NW = NC * NS
EPW = E // NW
G = 80
NCHUNK = EPW // G


def _pack_bf16(y):
    blk, c = y.shape
    a = lax.bitcast_convert_type(y[:, :c // 2], jnp.uint32)
    b = lax.bitcast_convert_type(y[:, c // 2:], jnp.uint32)
    rnd = jnp.uint32(0x8000)
    w = ((a + rnd) & jnp.uint32(0xFFFF0000)) | ((b + rnd) >> 16)
    return lax.bitcast_convert_type(w, jnp.float32)


def _unpack_bf16(p):
    w = lax.bitcast_convert_type(p, jnp.uint32)
    a = lax.bitcast_convert_type(w & jnp.uint32(0xFFFF0000), jnp.float32)
    b = lax.bitcast_convert_type(w << 16, jnp.float32)
    return jnp.concatenate([a, b], axis=1)


def _proj_body(x_ref, w_ref, b_ref, q_ref, k_ref, v_ref, s_ref):
    y = (jnp.dot(x_ref[...], w_ref[...], preferred_element_type=jnp.float32)
         + b_ref[...])
    q_ref[...] = _pack_bf16(y[:, 0 * HID:1 * HID])
    k_ref[...] = _pack_bf16(y[:, 1 * HID:2 * HID])
    v_ref[...] = _pack_bf16(y[:, 2 * HID:3 * HID])
    s_ref[...] = y[:, 3 * HID:4 * HID]


def _proj(x, w, b):
    n, fin = x.shape
    blk = ROW_BLK
    outp = jax.ShapeDtypeStruct((n, HID // 2), jnp.float32)
    return pl.pallas_call(
        _proj_body,
        grid=(n // blk,),
        in_specs=[
            pl.BlockSpec((blk, fin), lambda i: (i, 0)),
            pl.BlockSpec((fin, 4 * HID), lambda i: (0, 0)),
            pl.BlockSpec((1, 4 * HID), lambda i: (0, 0)),
        ],
        out_specs=[pl.BlockSpec((blk, HID // 2), lambda i: (i, 0))] * 3 +
                  [pl.BlockSpec((blk, HID), lambda i: (i, 0))],
        out_shape=[outp, outp, outp,
                   jax.ShapeDtypeStruct((n, HID), jnp.float32)],
    )(x, w, b.reshape(1, 4 * HID))


_sc_mesh = plsc.VectorSubcoreMesh(core_axis_name="c", subcore_axis_name="s")


@functools.partial(
    pl.kernel,
    mesh=_sc_mesh,
    out_type=[jax.ShapeDtypeStruct((E, HID // 2), jnp.float32)] * 3,
    scratch_types=[
        pltpu.VMEM((G,), jnp.int32),
        pltpu.VMEM((G,), jnp.int32),
        pltpu.VMEM((G, HID // 2), jnp.float32),
        pltpu.VMEM((G, HID // 2), jnp.float32),
        pltpu.SemaphoreType.DMA,
        pltpu.SemaphoreType.DMA,
    ],
)
def _sc_gather(q_hbm, k_hbm, v_hbm, src_hbm, dst_hbm,
               qd_hbm, ks_hbm, vs_hbm, idx0, idx1, rows0, rows1,
               sem0, sem1):
    wid = lax.axis_index("s") * NC + lax.axis_index("c")
    base = wid * EPW
    idxb = (idx0, idx1)
    rowsb = (rows0, rows1)
    semb = (sem0, sem1)

    def one_table(tab, idxarr, out):
        def start(i, b):
            pltpu.sync_copy(idxarr.at[pl.ds(base + i * G, G)], idxb[b])
            pltpu.async_copy(tab.at[idxb[b]], rowsb[b], semb[b])

        def drain(i, b):
            pltpu.make_async_copy(tab.at[idxb[b]], rowsb[b], semb[b]).wait()
            pltpu.sync_copy(rowsb[b], out.at[pl.ds(base + i * G, G)])

        start(0, 0)

        def pair(j, c):
            i1 = 2 * j + 1
            start(i1, 1)
            drain(i1 - 1, 0)
            start(i1 + 1, 0)
            drain(i1, 1)
            return c
        lax.fori_loop(0, (NCHUNK - 1) // 2, pair, 0)
        drain(NCHUNK - 1, (NCHUNK - 1) % 2)

    one_table(q_hbm, dst_hbm, qd_hbm)
    one_table(k_hbm, src_hbm, ks_hbm)
    one_table(v_hbm, src_hbm, vs_hbm)


def _edge_math_body(qd_ref, ks_ref, vs_ref,
                    m0_ref, m1_ref, m2_ref, m3_ref, m4_ref):
    blk = qd_ref.shape[0]
    prod = _unpack_bf16(qd_ref[...]) * _unpack_bf16(ks_ref[...])
    alpha = jnp.sum(prod.reshape(blk, HEADS, OUT_CH), axis=-1) * 0.125
    ex = jnp.exp(alpha)
    exfull = jnp.repeat(ex, OUT_CH, axis=1)
    m = _unpack_bf16(vs_ref[...]) * exfull
    m0_ref[...] = m[:, 0:128]
    m1_ref[...] = m[:, 128:256]
    m2_ref[...] = m[:, 256:384]
    m3_ref[...] = m[:, 384:512]
    m4_ref[...] = jnp.concatenate(
        [ex, jnp.zeros((blk, 120), jnp.float32)], axis=1)


def _edge_math(qd, ks, vs):
    blk = EDGE_BLK
    mout = jax.ShapeDtypeStruct((E, 128), jnp.float32)
    return pl.pallas_call(
        _edge_math_body,
        grid=(E // blk,),
        in_specs=[pl.BlockSpec((blk, HID // 2), lambda i: (i, 0))] * 3,
        out_specs=[pl.BlockSpec((blk, 128), lambda i: (i, 0))] * 5,
        out_shape=[mout, mout, mout, mout, mout],
    )(qd, ks, vs)


@functools.partial(
    pl.kernel,
    mesh=_sc_mesh,
    out_type=[jax.ShapeDtypeStruct((5, N, 128), jnp.float32),
              jax.ShapeDtypeStruct((5, N, 128), jnp.float32)],
    scratch_types=[
        pltpu.VMEM((G,), jnp.int32),
        pltpu.VMEM((G,), jnp.int32),
        pltpu.VMEM((G, 128), jnp.float32),
        pltpu.VMEM((G, 128), jnp.float32),
        pltpu.VMEM((G, 128), jnp.float32),
        pltpu.VMEM_SHARED((N, 128), jnp.float32),
        pltpu.SemaphoreType.DMA,
        pltpu.SemaphoreType.DMA,
    ],
)
def _sc_scatter(m0_hbm, m1_hbm, m2_hbm, m3_hbm, m4_hbm, dst_hbm, z128_hbm,
                agg0_hbm, agg1_hbm, idxa, idxb, mbufa, mbufb, zvb, acc,
                sema, semb):
    cid = lax.axis_index("c")
    sid = lax.axis_index("s")
    wid = sid * NC + cid
    base = wid * EPW

    pltpu.sync_copy(z128_hbm.at[pl.ds(0, G)], zvb)

    rstart = sid * 640
    rend = jnp.minimum(rstart + 640, N)

    def rowchunks(fn):
        for j in range(8):
            off = rstart + j * G
            @pl.when(off < rend)
            def _():
                fn(pl.ds(off, G))

    idxs = (idxa, idxb)
    mbufs = (mbufa, mbufb)
    sems = (sema, semb)

    for g, mg in enumerate((m0_hbm, m1_hbm, m2_hbm, m3_hbm, m4_hbm)):
        rowchunks(lambda r: pltpu.sync_copy(zvb, acc.at[r]))
        plsc.subcore_barrier()

        def start(i, b):
            pltpu.sync_copy(dst_hbm.at[pl.ds(base + i * G, G)], idxs[b])
            pltpu.async_copy(mg.at[pl.ds(base + i * G, G)], mbufs[b],
                             sems[b])

        def commit(i, b):
            pltpu.make_async_copy(mg.at[pl.ds(base + i * G, G)], mbufs[b],
                                  sems[b]).wait()
            pltpu.sync_copy(mbufs[b], acc.at[idxs[b]], add=True)

        start(0, 0)

        def pair(j, c):
            i1 = 2 * j + 1
            start(i1, 1)
            commit(i1 - 1, 0)
            start(i1 + 1, 0)
            commit(i1, 1)
            return c
        lax.fori_loop(0, (NCHUNK - 1) // 2, pair, 0)
        commit(NCHUNK - 1, (NCHUNK - 1) % 2)
        plsc.subcore_barrier()

        def flush(out):
            def one(r):
                pltpu.sync_copy(acc.at[r], mbufa)
                pltpu.sync_copy(mbufa, out.at[g, r])
            rowchunks(one)

        @pl.when(cid == 0)
        def _():
            flush(agg0_hbm)

        @pl.when(cid == 1)
        def _():
            flush(agg1_hbm)

        plsc.subcore_barrier()


def _epilogue_body(a00, a01, a02, a03, a04, a10, a11, a12, a13, a14,
                   skip_ref, res_ref, g_ref, b_ref, o_ref):
    agg = jnp.concatenate(
        [a00[...] + a10[...], a01[...] + a11[...],
         a02[...] + a12[...], a03[...] + a13[...]], axis=1)
    den8 = (a04[...] + a14[...])[:, 0:8]
    den_full = jnp.repeat(den8, OUT_CH, axis=1)
    h = agg / (den_full + 1e-16) + skip_ref[...]
    h = jnp.maximum(h, 0.0) + res_ref[...]
    mu = jnp.mean(h, axis=1, keepdims=True)
    var = jnp.mean((h - mu) ** 2, axis=1, keepdims=True)
    o_ref[...] = (h - mu) / jnp.sqrt(var + 1e-5) * g_ref[...] + b_ref[...]


def _epilogue(agg0, agg1, skip, res, g, b):
    blk = ROW_BLK
    aspec = [pl.BlockSpec((blk, 128), lambda i: (i, 0))] * 10
    return pl.pallas_call(
        _epilogue_body,
        grid=(N // blk,),
        in_specs=aspec + [
            pl.BlockSpec((blk, HID), lambda i: (i, 0)),
            pl.BlockSpec((blk, HID), lambda i: (i, 0)),
            pl.BlockSpec((1, HID), lambda i: (0, 0)),
            pl.BlockSpec((1, HID), lambda i: (0, 0)),
        ],
        out_specs=pl.BlockSpec((blk, HID), lambda i: (i, 0)),
        out_shape=jax.ShapeDtypeStruct((N, HID), jnp.float32),
    )(agg0[0], agg0[1], agg0[2], agg0[3], agg0[4],
      agg1[0], agg1[1], agg1[2], agg1[3], agg1[4],
      skip, res, g.reshape(1, HID), b.reshape(1, HID))


def _pool_body(h_ref, b_ref, sums_ref, cnt_ref):
    blk = h_ref.shape[0]
    oh = (b_ref[...] == lax.broadcasted_iota(jnp.int32, (1, NUM_GRAPHS), 1)
          ).astype(jnp.float32)
    part = lax.dot_general(oh, h_ref[...], (((0,), (0,)), ((), ())),
                           preferred_element_type=jnp.float32)
    cpart = lax.dot_general(oh, jnp.ones((blk, 128), jnp.float32),
                            (((0,), (0,)), ((), ())),
                            preferred_element_type=jnp.float32)

    @pl.when(pl.program_id(0) == 0)
    def _():
        sums_ref[...] = jnp.zeros_like(sums_ref)
        cnt_ref[...] = jnp.zeros_like(cnt_ref)

    sums_ref[...] += part
    cnt_ref[...] += cpart


def _pool(h, batch2):
    blk = ROW_BLK
    return pl.pallas_call(
        _pool_body,
        grid=(N // blk,),
        in_specs=[
            pl.BlockSpec((blk, HID), lambda i: (i, 0)),
            pl.BlockSpec((blk, 1), lambda i: (i, 0)),
        ],
        out_specs=[pl.BlockSpec((NUM_GRAPHS, HID), lambda i: (0, 0)),
                   pl.BlockSpec((NUM_GRAPHS, 128), lambda i: (0, 0))],
        out_shape=[jax.ShapeDtypeStruct((NUM_GRAPHS, HID), jnp.float32),
                   jax.ShapeDtypeStruct((NUM_GRAPHS, 128), jnp.float32)],
    )(h, batch2)


def _head_body(s_ref, c_ref, w1_ref, b1_ref, w2_ref, b2_ref, o_ref):
    cnt = jnp.maximum(c_ref[...], 1.0)
    graph = (s_ref[...].reshape(NUM_GRAPHS, 4, 128) / cnt[:, None, :]
             ).reshape(NUM_GRAPHS, HID)
    h = jnp.dot(graph, w1_ref[...], preferred_element_type=jnp.float32)
    h = jnp.maximum(h + b1_ref[...], 0.0)
    o_ref[...] = (jnp.dot(h, w2_ref[...], preferred_element_type=jnp.float32)
                  + b2_ref[...])


def _head(sums, cnt, hp):
    return pl.pallas_call(
        _head_body,
        grid=(1,),
        in_specs=[
            pl.BlockSpec((NUM_GRAPHS, HID), lambda i: (0, 0)),
            pl.BlockSpec((NUM_GRAPHS, 128), lambda i: (0, 0)),
            pl.BlockSpec((HID, OUT_CH), lambda i: (0, 0)),
            pl.BlockSpec((1, OUT_CH), lambda i: (0, 0)),
            pl.BlockSpec((OUT_CH, 1), lambda i: (0, 0)),
            pl.BlockSpec((1, 1), lambda i: (0, 0)),
        ],
        out_specs=pl.BlockSpec((NUM_GRAPHS, 1), lambda i: (0, 0)),
        out_shape=jax.ShapeDtypeStruct((NUM_GRAPHS, 1), jnp.float32),
    )(sums, cnt, hp["W1"], hp["b1"].reshape(1, OUT_CH), hp["W2"],
      hp["b2"].reshape(1, 1))


def kernel(x, params, edge_index, batch):
    src = edge_index[0]
    dst = edge_index[1]
    z128 = jnp.zeros((G, 128), jnp.float32)
    cs = params["convs"]
    h = x
    res = jnp.zeros((N, HID), jnp.float32)
    for l in range(4):
        p = cs[l]
        wall = jnp.concatenate([p["Wq"], p["Wk"], p["Wv"], p["Ws"]], axis=1)
        ball = jnp.concatenate([p["bq"], p["bk"], p["bv"], p["bs"]], axis=0)
        q, k, v, skip = _proj(h, wall, ball)
        qd, ks, vs = _sc_gather(q, k, v, src, dst)
        m0, m1, m2, m3, m4 = _edge_math(qd, ks, vs)
        agg0, agg1 = _sc_scatter(m0, m1, m2, m3, m4, dst, z128)
        h = _epilogue(agg0, agg1, skip, res, p["ln_g"], p["ln_b"])
        res = h
    sums, cnt = _pool(h, batch.reshape(N, 1))
    return _head(sums, cnt, params["head"])


# speedup vs baseline: 18.8393x; 1.0612x over previous
import functools

import jax
import jax.numpy as jnp
from jax import lax
from jax.experimental import pallas as pl
from jax.experimental.pallas import tpu as pltpu
from jax.experimental.pallas import tpu_sc as plsc

N = 10000
E = 320000
IN_CH = 128
HEADS = 8
OUT_CH = 64
HID = HEADS * OUT_CH
NUM_GRAPHS = 64

ROW_BLK = 1000
EDGE_BLK = 2000
NC = 2
NS = 16
NW = NC * NS
EPW = E // NW
G = 80
GB = 128
NBH = (EPW - 16) // GB
TAIL = EPW - NBH * GB


def _pack_bf16(y):
    blk, c = y.shape
    a = lax.bitcast_convert_type(y[:, :c // 2], jnp.uint32)
    b = lax.bitcast_convert_type(y[:, c // 2:], jnp.uint32)
    rnd = jnp.uint32(0x8000)
    w = ((a + rnd) & jnp.uint32(0xFFFF0000)) | ((b + rnd) >> 16)
    return lax.bitcast_convert_type(w, jnp.float32)


def _unpack_bf16(p):
    w = lax.bitcast_convert_type(p, jnp.uint32)
    a = lax.bitcast_convert_type(w & jnp.uint32(0xFFFF0000), jnp.float32)
    b = lax.bitcast_convert_type(w << 16, jnp.float32)
    return jnp.concatenate([a, b], axis=1)


def _proj_body(x_ref, w_ref, b_ref, q_ref, k_ref, v_ref, s_ref):
    y = (jnp.dot(x_ref[...], w_ref[...], preferred_element_type=jnp.float32)
         + b_ref[...])
    q_ref[...] = _pack_bf16(y[:, 0 * HID:1 * HID])
    k_ref[...] = _pack_bf16(y[:, 1 * HID:2 * HID])
    v_ref[...] = _pack_bf16(y[:, 2 * HID:3 * HID])
    s_ref[...] = y[:, 3 * HID:4 * HID]


def _proj(x, w, b):
    n, fin = x.shape
    blk = ROW_BLK
    outp = jax.ShapeDtypeStruct((n, HID // 2), jnp.float32)
    return pl.pallas_call(
        _proj_body,
        grid=(n // blk,),
        in_specs=[
            pl.BlockSpec((blk, fin), lambda i: (i, 0)),
            pl.BlockSpec((fin, 4 * HID), lambda i: (0, 0)),
            pl.BlockSpec((1, 4 * HID), lambda i: (0, 0)),
        ],
        out_specs=[pl.BlockSpec((blk, HID // 2), lambda i: (i, 0))] * 3 +
                  [pl.BlockSpec((blk, HID), lambda i: (i, 0))],
        out_shape=[outp, outp, outp,
                   jax.ShapeDtypeStruct((n, HID), jnp.float32)],
    )(x, w, b.reshape(1, 4 * HID))


_sc_mesh = plsc.VectorSubcoreMesh(core_axis_name="c", subcore_axis_name="s")


def _pipeline2(nch, start, drain):
    start(0, 0)

    def pair(j, c):
        i1 = 2 * j + 1
        start(i1, 1)
        drain(i1 - 1, 0)
        start(i1 + 1, 0)
        drain(i1, 1)
        return c
    lax.fori_loop(0, (nch - 1) // 2, pair, 0)
    if nch % 2 == 0:
        start(nch - 1, 1)
        drain(nch - 2, 0)
        drain(nch - 1, 1)
    else:
        drain(nch - 1, 0)


@functools.partial(
    pl.kernel,
    mesh=_sc_mesh,
    out_type=[jax.ShapeDtypeStruct((E, HID // 2), jnp.float32)] * 3,
    scratch_types=[
        pltpu.VMEM((GB,), jnp.int32),
        pltpu.VMEM((GB,), jnp.int32),
        pltpu.VMEM((TAIL,), jnp.int32),
        pltpu.VMEM((GB, HID // 2), jnp.float32),
        pltpu.VMEM((GB, HID // 2), jnp.float32),
        pltpu.SemaphoreType.DMA,
        pltpu.SemaphoreType.DMA,
    ],
)
def _sc_gather(q_hbm, k_hbm, v_hbm, src_hbm, dst_hbm,
               qd_hbm, ks_hbm, vs_hbm, idx0, idx1, idxt, rows0, rows1,
               sem0, sem1):
    wid = lax.axis_index("s") * NC + lax.axis_index("c")
    base = wid * EPW
    idxb = (idx0, idx1)
    rowsb = (rows0, rows1)
    semb = (sem0, sem1)

    def one_table(tab, idxarr, out):
        pltpu.sync_copy(idxarr.at[pl.ds(base, TAIL)], idxt)
        pltpu.async_copy(tab.at[idxt], rows0.at[pl.ds(0, TAIL)], sem0).wait()
        pltpu.sync_copy(rows0.at[pl.ds(0, TAIL)], out.at[pl.ds(base, TAIL)])

        def start(i, b):
            off = base + TAIL + i * GB
            pltpu.sync_copy(idxarr.at[pl.ds(off, GB)], idxb[b])
            pltpu.async_copy(tab.at[idxb[b]], rowsb[b], semb[b])

        def drain(i, b):
            off = base + TAIL + i * GB
            pltpu.make_async_copy(tab.at[idxb[b]], rowsb[b], semb[b]).wait()
            pltpu.sync_copy(rowsb[b], out.at[pl.ds(off, GB)])

        _pipeline2(NBH, start, drain)

    one_table(q_hbm, dst_hbm, qd_hbm)
    one_table(k_hbm, src_hbm, ks_hbm)
    one_table(v_hbm, src_hbm, vs_hbm)


def _edge_math_body(qd_ref, ks_ref, vs_ref,
                    m0_ref, m1_ref, m2_ref, m3_ref, m4_ref):
    blk = qd_ref.shape[0]
    prod = _unpack_bf16(qd_ref[...]) * _unpack_bf16(ks_ref[...])
    alpha = jnp.sum(prod.reshape(blk, HEADS, OUT_CH), axis=-1) * 0.125
    ex = jnp.exp(alpha)
    exfull = jnp.repeat(ex, OUT_CH, axis=1)
    m = _unpack_bf16(vs_ref[...]) * exfull
    m0_ref[...] = m[:, 0:128]
    m1_ref[...] = m[:, 128:256]
    m2_ref[...] = m[:, 256:384]
    m3_ref[...] = m[:, 384:512]
    m4_ref[...] = jnp.concatenate(
        [ex, jnp.zeros((blk, 120), jnp.float32)], axis=1)


def _edge_math(qd, ks, vs):
    blk = EDGE_BLK
    mout = jax.ShapeDtypeStruct((E, 128), jnp.float32)
    return pl.pallas_call(
        _edge_math_body,
        grid=(E // blk,),
        in_specs=[pl.BlockSpec((blk, HID // 2), lambda i: (i, 0))] * 3,
        out_specs=[pl.BlockSpec((blk, 128), lambda i: (i, 0))] * 5,
        out_shape=[mout, mout, mout, mout, mout],
    )(qd, ks, vs)


@functools.partial(
    pl.kernel,
    mesh=_sc_mesh,
    out_type=[jax.ShapeDtypeStruct((5, N, 128), jnp.float32),
              jax.ShapeDtypeStruct((5, N, 128), jnp.float32)],
    scratch_types=[
        pltpu.VMEM((GB,), jnp.int32),
        pltpu.VMEM((GB,), jnp.int32),
        pltpu.VMEM((TAIL,), jnp.int32),
        pltpu.VMEM((GB, 128), jnp.float32),
        pltpu.VMEM((GB, 128), jnp.float32),
        pltpu.VMEM((G, 128), jnp.float32),
        pltpu.VMEM_SHARED((N, 128), jnp.float32),
        pltpu.SemaphoreType.DMA,
        pltpu.SemaphoreType.DMA,
    ],
)
def _sc_scatter(m0_hbm, m1_hbm, m2_hbm, m3_hbm, m4_hbm, dst_hbm, z128_hbm,
                agg0_hbm, agg1_hbm, idxa, idxb, idxt, mbufa, mbufb, zvb, acc,
                sema, semb):
    cid = lax.axis_index("c")
    sid = lax.axis_index("s")
    wid = sid * NC + cid
    base = wid * EPW

    pltpu.sync_copy(z128_hbm.at[pl.ds(0, G)], zvb)

    rstart = sid * 640
    rend = jnp.minimum(rstart + 640, N)

    def rowchunks(fn):
        for j in range(8):
            off = rstart + j * G
            @pl.when(off < rend)
            def _():
                fn(pl.ds(off, G))

    idxs = (idxa, idxb)
    sems = (sema, semb)

    for g, mg in enumerate((m0_hbm, m1_hbm, m2_hbm, m3_hbm, m4_hbm)):
        rowchunks(lambda r: pltpu.sync_copy(zvb, acc.at[r]))
        plsc.subcore_barrier()

        pltpu.sync_copy(dst_hbm.at[pl.ds(base, TAIL)], idxt)
        pltpu.sync_copy(mg.at[pl.ds(base, TAIL)], mbufa.at[pl.ds(0, TAIL)])
        pltpu.sync_copy(mbufa.at[pl.ds(0, TAIL)], acc.at[idxt], add=True)

        mbufs = (mbufa, mbufb)

        def start(i, b):
            off = base + TAIL + i * GB
            pltpu.sync_copy(dst_hbm.at[pl.ds(off, GB)], idxs[b])
            pltpu.async_copy(mg.at[pl.ds(off, GB)], mbufs[b], sems[b])

        def commit(i, b):
            off = base + TAIL + i * GB
            pltpu.make_async_copy(mg.at[pl.ds(off, GB)], mbufs[b],
                                  sems[b]).wait()
            pltpu.sync_copy(mbufs[b], acc.at[idxs[b]], add=True)

        _pipeline2(NBH, start, commit)
        plsc.subcore_barrier()

        def flush(out):
            def one(r):
                pltpu.sync_copy(acc.at[r], mbufa.at[pl.ds(0, G)])
                pltpu.sync_copy(mbufa.at[pl.ds(0, G)], out.at[g, r])
            rowchunks(one)

        @pl.when(cid == 0)
        def _():
            flush(agg0_hbm)

        @pl.when(cid == 1)
        def _():
            flush(agg1_hbm)

        plsc.subcore_barrier()


def _epilogue_body(a00, a01, a02, a03, a04, a10, a11, a12, a13, a14,
                   skip_ref, res_ref, g_ref, b_ref, o_ref):
    agg = jnp.concatenate(
        [a00[...] + a10[...], a01[...] + a11[...],
         a02[...] + a12[...], a03[...] + a13[...]], axis=1)
    den8 = (a04[...] + a14[...])[:, 0:8]
    den_full = jnp.repeat(den8, OUT_CH, axis=1)
    h = agg / (den_full + 1e-16) + skip_ref[...]
    h = jnp.maximum(h, 0.0) + res_ref[...]
    mu = jnp.mean(h, axis=1, keepdims=True)
    var = jnp.mean((h - mu) ** 2, axis=1, keepdims=True)
    o_ref[...] = (h - mu) / jnp.sqrt(var + 1e-5) * g_ref[...] + b_ref[...]


def _epilogue(agg0, agg1, skip, res, g, b):
    blk = ROW_BLK
    aspec = [pl.BlockSpec((blk, 128), lambda i: (i, 0))] * 10
    return pl.pallas_call(
        _epilogue_body,
        grid=(N // blk,),
        in_specs=aspec + [
            pl.BlockSpec((blk, HID), lambda i: (i, 0)),
            pl.BlockSpec((blk, HID), lambda i: (i, 0)),
            pl.BlockSpec((1, HID), lambda i: (0, 0)),
            pl.BlockSpec((1, HID), lambda i: (0, 0)),
        ],
        out_specs=pl.BlockSpec((blk, HID), lambda i: (i, 0)),
        out_shape=jax.ShapeDtypeStruct((N, HID), jnp.float32),
    )(agg0[0], agg0[1], agg0[2], agg0[3], agg0[4],
      agg1[0], agg1[1], agg1[2], agg1[3], agg1[4],
      skip, res, g.reshape(1, HID), b.reshape(1, HID))


def _pool_body(h_ref, b_ref, sums_ref, cnt_ref):
    blk = h_ref.shape[0]
    oh = (b_ref[...] == lax.broadcasted_iota(jnp.int32, (1, NUM_GRAPHS), 1)
          ).astype(jnp.float32)
    part = lax.dot_general(oh, h_ref[...], (((0,), (0,)), ((), ())),
                           preferred_element_type=jnp.float32)
    cpart = lax.dot_general(oh, jnp.ones((blk, 128), jnp.float32),
                            (((0,), (0,)), ((), ())),
                            preferred_element_type=jnp.float32)

    @pl.when(pl.program_id(0) == 0)
    def _():
        sums_ref[...] = jnp.zeros_like(sums_ref)
        cnt_ref[...] = jnp.zeros_like(cnt_ref)

    sums_ref[...] += part
    cnt_ref[...] += cpart


def _pool(h, batch2):
    blk = ROW_BLK
    return pl.pallas_call(
        _pool_body,
        grid=(N // blk,),
        in_specs=[
            pl.BlockSpec((blk, HID), lambda i: (i, 0)),
            pl.BlockSpec((blk, 1), lambda i: (i, 0)),
        ],
        out_specs=[pl.BlockSpec((NUM_GRAPHS, HID), lambda i: (0, 0)),
                   pl.BlockSpec((NUM_GRAPHS, 128), lambda i: (0, 0))],
        out_shape=[jax.ShapeDtypeStruct((NUM_GRAPHS, HID), jnp.float32),
                   jax.ShapeDtypeStruct((NUM_GRAPHS, 128), jnp.float32)],
    )(h, batch2)


def _head_body(s_ref, c_ref, w1_ref, b1_ref, w2_ref, b2_ref, o_ref):
    cnt = jnp.maximum(c_ref[...], 1.0)
    graph = (s_ref[...].reshape(NUM_GRAPHS, 4, 128) / cnt[:, None, :]
             ).reshape(NUM_GRAPHS, HID)
    h = jnp.dot(graph, w1_ref[...], preferred_element_type=jnp.float32)
    h = jnp.maximum(h + b1_ref[...], 0.0)
    o_ref[...] = (jnp.dot(h, w2_ref[...], preferred_element_type=jnp.float32)
                  + b2_ref[...])


def _head(sums, cnt, hp):
    return pl.pallas_call(
        _head_body,
        grid=(1,),
        in_specs=[
            pl.BlockSpec((NUM_GRAPHS, HID), lambda i: (0, 0)),
            pl.BlockSpec((NUM_GRAPHS, 128), lambda i: (0, 0)),
            pl.BlockSpec((HID, OUT_CH), lambda i: (0, 0)),
            pl.BlockSpec((1, OUT_CH), lambda i: (0, 0)),
            pl.BlockSpec((OUT_CH, 1), lambda i: (0, 0)),
            pl.BlockSpec((1, 1), lambda i: (0, 0)),
        ],
        out_specs=pl.BlockSpec((NUM_GRAPHS, 1), lambda i: (0, 0)),
        out_shape=jax.ShapeDtypeStruct((NUM_GRAPHS, 1), jnp.float32),
    )(sums, cnt, hp["W1"], hp["b1"].reshape(1, OUT_CH), hp["W2"],
      hp["b2"].reshape(1, 1))


def kernel(x, params, edge_index, batch):
    src = edge_index[0]
    dst = edge_index[1]
    z128 = jnp.zeros((G, 128), jnp.float32)
    cs = params["convs"]
    h = x
    res = jnp.zeros((N, HID), jnp.float32)
    for l in range(4):
        p = cs[l]
        wall = jnp.concatenate([p["Wq"], p["Wk"], p["Wv"], p["Ws"]], axis=1)
        ball = jnp.concatenate([p["bq"], p["bk"], p["bv"], p["bs"]], axis=0)
        q, k, v, skip = _proj(h, wall, ball)
        qd, ks, vs = _sc_gather(q, k, v, src, dst)
        m0, m1, m2, m3, m4 = _edge_math(qd, ks, vs)
        agg0, agg1 = _sc_scatter(m0, m1, m2, m3, m4, dst, z128)
        h = _epilogue(agg0, agg1, skip, res, p["ln_g"], p["ln_b"])
        res = h
    sums, cnt = _pool(h, batch.reshape(N, 1))
    return _head(sums, cnt, params["head"])


# speedup vs baseline: 20.2853x; 1.0768x over previous
import functools

import jax
import jax.numpy as jnp
from jax import lax
from jax.experimental import pallas as pl
from jax.experimental.pallas import tpu as pltpu
from jax.experimental.pallas import tpu_sc as plsc

N = 10000
E = 320000
IN_CH = 128
HEADS = 8
OUT_CH = 64
HID = HEADS * OUT_CH
NUM_GRAPHS = 64

ROW_BLK = 1000
EDGE_BLK = 2000
NC = 2
NS = 16
NW = NC * NS
E2 = E // 2
EPW = E2 // NW
G = 80
GB = 128
NBH = (EPW - 8) // GB
TAIL = EPW - NBH * GB


def _pack_bf16(y):
    blk, c = y.shape
    a = lax.bitcast_convert_type(y[:, :c // 2], jnp.uint32)
    b = lax.bitcast_convert_type(y[:, c // 2:], jnp.uint32)
    rnd = jnp.uint32(0x8000)
    w = ((a + rnd) & jnp.uint32(0xFFFF0000)) | ((b + rnd) >> 16)
    return lax.bitcast_convert_type(w, jnp.float32)


def _unpack_bf16(p):
    w = lax.bitcast_convert_type(p, jnp.uint32)
    a = lax.bitcast_convert_type(w & jnp.uint32(0xFFFF0000), jnp.float32)
    b = lax.bitcast_convert_type(w << 16, jnp.float32)
    return jnp.concatenate([a, b], axis=1)


def _proj_body(x_ref, w_ref, b_ref, q_ref, k_ref, v_ref, s_ref):
    y = (jnp.dot(x_ref[...], w_ref[...], preferred_element_type=jnp.float32)
         + b_ref[...])
    q_ref[...] = _pack_bf16(y[:, 0 * HID:1 * HID])
    k_ref[...] = _pack_bf16(y[:, 1 * HID:2 * HID])
    v_ref[...] = _pack_bf16(y[:, 2 * HID:3 * HID])
    s_ref[...] = y[:, 3 * HID:4 * HID]


def _proj(x, w, b):
    n, fin = x.shape
    blk = ROW_BLK
    outp = jax.ShapeDtypeStruct((n, HID // 2), jnp.float32)
    return pl.pallas_call(
        _proj_body,
        grid=(n // blk,),
        in_specs=[
            pl.BlockSpec((blk, fin), lambda i: (i, 0)),
            pl.BlockSpec((fin, 4 * HID), lambda i: (0, 0)),
            pl.BlockSpec((1, 4 * HID), lambda i: (0, 0)),
        ],
        out_specs=[pl.BlockSpec((blk, HID // 2), lambda i: (i, 0))] * 3 +
                  [pl.BlockSpec((blk, HID), lambda i: (i, 0))],
        out_shape=[outp, outp, outp,
                   jax.ShapeDtypeStruct((n, HID), jnp.float32)],
    )(x, w, b.reshape(1, 4 * HID))


_sc_mesh = plsc.VectorSubcoreMesh(core_axis_name="c", subcore_axis_name="s")


def _pipeline2(nch, start, drain):
    start(0, 0)

    def pair(j, c):
        i1 = 2 * j + 1
        start(i1, 1)
        drain(i1 - 1, 0)
        start(i1 + 1, 0)
        drain(i1, 1)
        return c
    lax.fori_loop(0, (nch - 1) // 2, pair, 0)
    if nch % 2 == 0:
        start(nch - 1, 1)
        drain(nch - 2, 0)
        drain(nch - 1, 1)
    else:
        drain(nch - 1, 0)


@functools.partial(
    pl.kernel,
    mesh=_sc_mesh,
    out_type=[jax.ShapeDtypeStruct((E2, HID // 2), jnp.float32)] * 3,
    scratch_types=[
        pltpu.VMEM((GB,), jnp.int32),
        pltpu.VMEM((GB,), jnp.int32),
        pltpu.VMEM((TAIL,), jnp.int32),
        pltpu.VMEM((GB, HID // 2), jnp.float32),
        pltpu.VMEM((GB, HID // 2), jnp.float32),
        pltpu.SemaphoreType.DMA,
        pltpu.SemaphoreType.DMA,
    ],
)
def _sc_gather(q_hbm, k_hbm, v_hbm, src_hbm, dst_hbm,
               qd_hbm, ks_hbm, vs_hbm, idx0, idx1, idxt, rows0, rows1,
               sem0, sem1):
    wid = lax.axis_index("s") * NC + lax.axis_index("c")
    base = wid * EPW
    idxb = (idx0, idx1)
    rowsb = (rows0, rows1)
    semb = (sem0, sem1)

    def one_table(tab, idxarr, out):
        pltpu.sync_copy(idxarr.at[pl.ds(base, TAIL)], idxt)
        pltpu.async_copy(tab.at[idxt], rows0.at[pl.ds(0, TAIL)], sem0).wait()
        pltpu.sync_copy(rows0.at[pl.ds(0, TAIL)], out.at[pl.ds(base, TAIL)])

        def start(i, b):
            off = base + TAIL + i * GB
            pltpu.sync_copy(idxarr.at[pl.ds(off, GB)], idxb[b])
            pltpu.async_copy(tab.at[idxb[b]], rowsb[b], semb[b])

        def drain(i, b):
            off = base + TAIL + i * GB
            pltpu.make_async_copy(tab.at[idxb[b]], rowsb[b], semb[b]).wait()
            pltpu.sync_copy(rowsb[b], out.at[pl.ds(off, GB)])

        _pipeline2(NBH, start, drain)

    one_table(q_hbm, dst_hbm, qd_hbm)
    one_table(k_hbm, src_hbm, ks_hbm)
    one_table(v_hbm, src_hbm, vs_hbm)


def _edge_math_body(qd_ref, ks_ref, vs_ref,
                    m0_ref, m1_ref, m2_ref, m3_ref, m4_ref):
    blk = qd_ref.shape[0]
    prod = _unpack_bf16(qd_ref[...]) * _unpack_bf16(ks_ref[...])
    alpha = jnp.sum(prod.reshape(blk, HEADS, OUT_CH), axis=-1) * 0.125
    ex = jnp.exp(alpha)
    exfull = jnp.repeat(ex, OUT_CH, axis=1)
    m = _unpack_bf16(vs_ref[...]) * exfull
    m0_ref[...] = m[:, 0:128]
    m1_ref[...] = m[:, 128:256]
    m2_ref[...] = m[:, 256:384]
    m3_ref[...] = m[:, 384:512]
    m4_ref[...] = jnp.concatenate(
        [ex, jnp.zeros((blk, 120), jnp.float32)], axis=1)


def _edge_math(qd, ks, vs):
    blk = EDGE_BLK
    mout = jax.ShapeDtypeStruct((E2, 128), jnp.float32)
    return pl.pallas_call(
        _edge_math_body,
        grid=(E2 // blk,),
        in_specs=[pl.BlockSpec((blk, HID // 2), lambda i: (i, 0))] * 3,
        out_specs=[pl.BlockSpec((blk, 128), lambda i: (i, 0))] * 5,
        out_shape=[mout, mout, mout, mout, mout],
    )(qd, ks, vs)


@functools.partial(
    pl.kernel,
    mesh=_sc_mesh,
    out_type=[jax.ShapeDtypeStruct((5, N, 128), jnp.float32),
              jax.ShapeDtypeStruct((5, N, 128), jnp.float32)],
    scratch_types=[
        pltpu.VMEM((GB,), jnp.int32),
        pltpu.VMEM((GB,), jnp.int32),
        pltpu.VMEM((TAIL,), jnp.int32),
        pltpu.VMEM((GB, 128), jnp.float32),
        pltpu.VMEM((GB, 128), jnp.float32),
        pltpu.VMEM((G, 128), jnp.float32),
        pltpu.VMEM_SHARED((N, 128), jnp.float32),
        pltpu.SemaphoreType.DMA,
        pltpu.SemaphoreType.DMA,
    ],
)
def _sc_scatter(ma0, ma1, ma2, ma3, ma4, mb0, mb1, mb2, mb3, mb4,
                dsta_hbm, dstb_hbm, z128_hbm,
                agg0_hbm, agg1_hbm, idxa, idxb, idxt, mbufa, mbufb, zvb, acc,
                sema, semb):
    cid = lax.axis_index("c")
    sid = lax.axis_index("s")
    wid = sid * NC + cid
    base = wid * EPW

    pltpu.sync_copy(z128_hbm.at[pl.ds(0, G)], zvb)

    rstart = sid * 640
    rend = jnp.minimum(rstart + 640, N)

    def rowchunks(fn):
        for j in range(8):
            off = rstart + j * G
            @pl.when(off < rend)
            def _():
                fn(pl.ds(off, G))

    idxs = (idxa, idxb)
    sems = (sema, semb)

    halves = (((ma0, ma1, ma2, ma3, ma4), dsta_hbm),
              ((mb0, mb1, mb2, mb3, mb4), dstb_hbm))
    mbufs = (mbufa, mbufb)

    for g in range(5):
        rowchunks(lambda r: pltpu.sync_copy(zvb, acc.at[r]))
        plsc.subcore_barrier()

        for ms, dst_hbm in halves:
            mg = ms[g]
            pltpu.sync_copy(dst_hbm.at[pl.ds(base, TAIL)], idxt)
            pltpu.sync_copy(mg.at[pl.ds(base, TAIL)],
                            mbufa.at[pl.ds(0, TAIL)])
            pltpu.sync_copy(mbufa.at[pl.ds(0, TAIL)], acc.at[idxt],
                            add=True)

            def start(i, b):
                off = base + TAIL + i * GB
                pltpu.sync_copy(dst_hbm.at[pl.ds(off, GB)], idxs[b])
                pltpu.async_copy(mg.at[pl.ds(off, GB)], mbufs[b], sems[b])

            def commit(i, b):
                off = base + TAIL + i * GB
                pltpu.make_async_copy(mg.at[pl.ds(off, GB)], mbufs[b],
                                      sems[b]).wait()
                pltpu.sync_copy(mbufs[b], acc.at[idxs[b]], add=True)

            _pipeline2(NBH, start, commit)
        plsc.subcore_barrier()

        def flush(out):
            def one(r):
                pltpu.sync_copy(acc.at[r], mbufa.at[pl.ds(0, G)])
                pltpu.sync_copy(mbufa.at[pl.ds(0, G)], out.at[g, r])
            rowchunks(one)

        @pl.when(cid == 0)
        def _():
            flush(agg0_hbm)

        @pl.when(cid == 1)
        def _():
            flush(agg1_hbm)

        plsc.subcore_barrier()


def _epilogue_body(a00, a01, a02, a03, a04, a10, a11, a12, a13, a14,
                   skip_ref, res_ref, g_ref, b_ref, o_ref):
    agg = jnp.concatenate(
        [a00[...] + a10[...], a01[...] + a11[...],
         a02[...] + a12[...], a03[...] + a13[...]], axis=1)
    den8 = (a04[...] + a14[...])[:, 0:8]
    den_full = jnp.repeat(den8, OUT_CH, axis=1)
    h = agg / (den_full + 1e-16) + skip_ref[...]
    h = jnp.maximum(h, 0.0) + res_ref[...]
    mu = jnp.mean(h, axis=1, keepdims=True)
    var = jnp.mean((h - mu) ** 2, axis=1, keepdims=True)
    o_ref[...] = (h - mu) / jnp.sqrt(var + 1e-5) * g_ref[...] + b_ref[...]


def _epilogue(agg0, agg1, skip, res, g, b):
    blk = ROW_BLK
    aspec = [pl.BlockSpec((blk, 128), lambda i: (i, 0))] * 10
    return pl.pallas_call(
        _epilogue_body,
        grid=(N // blk,),
        in_specs=aspec + [
            pl.BlockSpec((blk, HID), lambda i: (i, 0)),
            pl.BlockSpec((blk, HID), lambda i: (i, 0)),
            pl.BlockSpec((1, HID), lambda i: (0, 0)),
            pl.BlockSpec((1, HID), lambda i: (0, 0)),
        ],
        out_specs=pl.BlockSpec((blk, HID), lambda i: (i, 0)),
        out_shape=jax.ShapeDtypeStruct((N, HID), jnp.float32),
    )(agg0[0], agg0[1], agg0[2], agg0[3], agg0[4],
      agg1[0], agg1[1], agg1[2], agg1[3], agg1[4],
      skip, res, g.reshape(1, HID), b.reshape(1, HID))


def _pool_body(h_ref, b_ref, sums_ref, cnt_ref):
    blk = h_ref.shape[0]
    oh = (b_ref[...] == lax.broadcasted_iota(jnp.int32, (1, NUM_GRAPHS), 1)
          ).astype(jnp.float32)
    part = lax.dot_general(oh, h_ref[...], (((0,), (0,)), ((), ())),
                           preferred_element_type=jnp.float32)
    cpart = lax.dot_general(oh, jnp.ones((blk, 128), jnp.float32),
                            (((0,), (0,)), ((), ())),
                            preferred_element_type=jnp.float32)

    @pl.when(pl.program_id(0) == 0)
    def _():
        sums_ref[...] = jnp.zeros_like(sums_ref)
        cnt_ref[...] = jnp.zeros_like(cnt_ref)

    sums_ref[...] += part
    cnt_ref[...] += cpart


def _pool(h, batch2):
    blk = ROW_BLK
    return pl.pallas_call(
        _pool_body,
        grid=(N // blk,),
        in_specs=[
            pl.BlockSpec((blk, HID), lambda i: (i, 0)),
            pl.BlockSpec((blk, 1), lambda i: (i, 0)),
        ],
        out_specs=[pl.BlockSpec((NUM_GRAPHS, HID), lambda i: (0, 0)),
                   pl.BlockSpec((NUM_GRAPHS, 128), lambda i: (0, 0))],
        out_shape=[jax.ShapeDtypeStruct((NUM_GRAPHS, HID), jnp.float32),
                   jax.ShapeDtypeStruct((NUM_GRAPHS, 128), jnp.float32)],
    )(h, batch2)


def _head_body(s_ref, c_ref, w1_ref, b1_ref, w2_ref, b2_ref, o_ref):
    cnt = jnp.maximum(c_ref[...], 1.0)
    graph = (s_ref[...].reshape(NUM_GRAPHS, 4, 128) / cnt[:, None, :]
             ).reshape(NUM_GRAPHS, HID)
    h = jnp.dot(graph, w1_ref[...], preferred_element_type=jnp.float32)
    h = jnp.maximum(h + b1_ref[...], 0.0)
    o_ref[...] = (jnp.dot(h, w2_ref[...], preferred_element_type=jnp.float32)
                  + b2_ref[...])


def _head(sums, cnt, hp):
    return pl.pallas_call(
        _head_body,
        grid=(1,),
        in_specs=[
            pl.BlockSpec((NUM_GRAPHS, HID), lambda i: (0, 0)),
            pl.BlockSpec((NUM_GRAPHS, 128), lambda i: (0, 0)),
            pl.BlockSpec((HID, OUT_CH), lambda i: (0, 0)),
            pl.BlockSpec((1, OUT_CH), lambda i: (0, 0)),
            pl.BlockSpec((OUT_CH, 1), lambda i: (0, 0)),
            pl.BlockSpec((1, 1), lambda i: (0, 0)),
        ],
        out_specs=pl.BlockSpec((NUM_GRAPHS, 1), lambda i: (0, 0)),
        out_shape=jax.ShapeDtypeStruct((NUM_GRAPHS, 1), jnp.float32),
    )(sums, cnt, hp["W1"], hp["b1"].reshape(1, OUT_CH), hp["W2"],
      hp["b2"].reshape(1, 1))


def kernel(x, params, edge_index, batch):
    src_a, src_b = edge_index[0, :E2], edge_index[0, E2:]
    dst_a, dst_b = edge_index[1, :E2], edge_index[1, E2:]
    z128 = jnp.zeros((G, 128), jnp.float32)
    cs = params["convs"]
    h = x
    res = jnp.zeros((N, HID), jnp.float32)
    for l in range(4):
        p = cs[l]
        wall = jnp.concatenate([p["Wq"], p["Wk"], p["Wv"], p["Ws"]], axis=1)
        ball = jnp.concatenate([p["bq"], p["bk"], p["bv"], p["bs"]], axis=0)
        q, k, v, skip = _proj(h, wall, ball)
        qda, ksa, vsa = _sc_gather(q, k, v, src_a, dst_a)
        ma = _edge_math(qda, ksa, vsa)
        qdb, ksb, vsb = _sc_gather(q, k, v, src_b, dst_b)
        mb = _edge_math(qdb, ksb, vsb)
        agg0, agg1 = _sc_scatter(*ma, *mb, dst_a, dst_b, z128)
        h = _epilogue(agg0, agg1, skip, res, p["ln_g"], p["ln_b"])
        res = h
    sums, cnt = _pool(h, batch.reshape(N, 1))
    return _head(sums, cnt, params["head"])


# speedup vs baseline: 21.0886x; 1.0396x over previous
import functools

import jax
import jax.numpy as jnp
from jax import lax
from jax.experimental import pallas as pl
from jax.experimental.pallas import tpu as pltpu
from jax.experimental.pallas import tpu_sc as plsc

N = 10000
E = 320000
IN_CH = 128
HEADS = 8
OUT_CH = 64
HID = HEADS * OUT_CH
NUM_GRAPHS = 64

ROW_BLK = 1000
EDGE_BLK = 2000
NC = 2
NS = 16
NW = NC * NS
E2 = E // 2
EPW = E2 // NW
G = 80
GB = 128
NBH = (EPW - 8) // GB
TAIL = EPW - NBH * GB


def _pack_bf16(y):
    blk, c = y.shape
    a = lax.bitcast_convert_type(y[:, :c // 2], jnp.uint32)
    b = lax.bitcast_convert_type(y[:, c // 2:], jnp.uint32)
    rnd = jnp.uint32(0x8000)
    w = ((a + rnd) & jnp.uint32(0xFFFF0000)) | ((b + rnd) >> 16)
    return lax.bitcast_convert_type(w, jnp.float32)


def _unpack_bf16(p):
    w = lax.bitcast_convert_type(p, jnp.uint32)
    a = lax.bitcast_convert_type(w & jnp.uint32(0xFFFF0000), jnp.float32)
    b = lax.bitcast_convert_type(w << 16, jnp.float32)
    return jnp.concatenate([a, b], axis=1)


def _proj_body(x_ref, w_ref, b_ref, q_ref, k_ref, v_ref, s_ref):
    y = (jnp.dot(x_ref[...], w_ref[...], preferred_element_type=jnp.float32)
         + b_ref[...])
    q_ref[...] = _pack_bf16(y[:, 0 * HID:1 * HID])
    k_ref[...] = _pack_bf16(y[:, 1 * HID:2 * HID])
    v_ref[...] = _pack_bf16(y[:, 2 * HID:3 * HID])
    s_ref[...] = y[:, 3 * HID:4 * HID]


def _proj(x, w, b):
    n, fin = x.shape
    blk = ROW_BLK
    outp = jax.ShapeDtypeStruct((n, HID // 2), jnp.float32)
    return pl.pallas_call(
        _proj_body,
        grid=(n // blk,),
        in_specs=[
            pl.BlockSpec((blk, fin), lambda i: (i, 0)),
            pl.BlockSpec((fin, 4 * HID), lambda i: (0, 0)),
            pl.BlockSpec((1, 4 * HID), lambda i: (0, 0)),
        ],
        out_specs=[pl.BlockSpec((blk, HID // 2), lambda i: (i, 0))] * 3 +
                  [pl.BlockSpec((blk, HID), lambda i: (i, 0))],
        out_shape=[outp, outp, outp,
                   jax.ShapeDtypeStruct((n, HID), jnp.float32)],
    )(x, w, b.reshape(1, 4 * HID))


_sc_mesh = plsc.VectorSubcoreMesh(core_axis_name="c", subcore_axis_name="s")


def _pipeline2(nch, start, drain):
    start(0, 0)

    def pair(j, c):
        i1 = 2 * j + 1
        start(i1, 1)
        drain(i1 - 1, 0)
        start(i1 + 1, 0)
        drain(i1, 1)
        return c
    lax.fori_loop(0, (nch - 1) // 2, pair, 0)
    if nch % 2 == 0:
        start(nch - 1, 1)
        drain(nch - 2, 0)
        drain(nch - 1, 1)
    else:
        drain(nch - 1, 0)


@functools.partial(
    pl.kernel,
    mesh=_sc_mesh,
    out_type=[jax.ShapeDtypeStruct((E2, HID // 2), jnp.float32)] * 3,
    scratch_types=[
        pltpu.VMEM((EPW,), jnp.int32),
        pltpu.VMEM((GB, HID // 2), jnp.float32),
        pltpu.VMEM((GB, HID // 2), jnp.float32),
        pltpu.SemaphoreType.DMA,
        pltpu.SemaphoreType.DMA,
    ],
)
def _sc_gather(q_hbm, k_hbm, v_hbm, src_hbm, dst_hbm,
               qd_hbm, ks_hbm, vs_hbm, idx_all, rows0, rows1, sem0, sem1):
    wid = lax.axis_index("s") * NC + lax.axis_index("c")
    base = wid * EPW
    rowsb = (rows0, rows1)
    semb = (sem0, sem1)

    def load_idx(idxarr):
        pltpu.sync_copy(idxarr.at[pl.ds(base, EPW)], idx_all)

    def one_table(tab, out):
        pltpu.async_copy(tab.at[idx_all.at[pl.ds(0, TAIL)]],
                         rows0.at[pl.ds(0, TAIL)], sem0).wait()
        pltpu.sync_copy(rows0.at[pl.ds(0, TAIL)], out.at[pl.ds(base, TAIL)])

        def start(i, b):
            pltpu.async_copy(tab.at[idx_all.at[pl.ds(TAIL + i * GB, GB)]],
                             rowsb[b], semb[b])

        def drain(i, b):
            pltpu.make_async_copy(
                tab.at[idx_all.at[pl.ds(TAIL + i * GB, GB)]],
                rowsb[b], semb[b]).wait()
            pltpu.sync_copy(rowsb[b], out.at[pl.ds(base + TAIL + i * GB, GB)])

        _pipeline2(NBH, start, drain)

    load_idx(dst_hbm)
    one_table(q_hbm, qd_hbm)
    load_idx(src_hbm)
    one_table(k_hbm, ks_hbm)
    one_table(v_hbm, vs_hbm)


def _edge_math_body(qd_ref, ks_ref, vs_ref,
                    m0_ref, m1_ref, m2_ref, m3_ref, m4_ref):
    blk = qd_ref.shape[0]
    prod = _unpack_bf16(qd_ref[...]) * _unpack_bf16(ks_ref[...])
    alpha = jnp.sum(prod.reshape(blk, HEADS, OUT_CH), axis=-1) * 0.125
    ex = jnp.exp(alpha)
    exfull = jnp.repeat(ex, OUT_CH, axis=1)
    m = _unpack_bf16(vs_ref[...]) * exfull
    m0_ref[...] = m[:, 0:128]
    m1_ref[...] = m[:, 128:256]
    m2_ref[...] = m[:, 256:384]
    m3_ref[...] = m[:, 384:512]
    m4_ref[...] = jnp.concatenate(
        [ex, jnp.zeros((blk, 120), jnp.float32)], axis=1)


def _edge_math(qd, ks, vs):
    blk = EDGE_BLK
    mout = jax.ShapeDtypeStruct((E2, 128), jnp.float32)
    return pl.pallas_call(
        _edge_math_body,
        grid=(E2 // blk,),
        in_specs=[pl.BlockSpec((blk, HID // 2), lambda i: (i, 0))] * 3,
        out_specs=[pl.BlockSpec((blk, 128), lambda i: (i, 0))] * 5,
        out_shape=[mout, mout, mout, mout, mout],
    )(qd, ks, vs)


@functools.partial(
    pl.kernel,
    mesh=_sc_mesh,
    out_type=[jax.ShapeDtypeStruct((5, N, 128), jnp.float32),
              jax.ShapeDtypeStruct((5, N, 128), jnp.float32)],
    scratch_types=[
        pltpu.VMEM((GB,), jnp.int32),
        pltpu.VMEM((GB,), jnp.int32),
        pltpu.VMEM((TAIL,), jnp.int32),
        pltpu.VMEM((GB, 128), jnp.float32),
        pltpu.VMEM((GB, 128), jnp.float32),
        pltpu.VMEM((G, 128), jnp.float32),
        pltpu.VMEM_SHARED((N, 128), jnp.float32),
        pltpu.SemaphoreType.DMA,
        pltpu.SemaphoreType.DMA,
        pltpu.SemaphoreType.DMA,
        pltpu.SemaphoreType.DMA,
    ],
)
def _sc_scatter(ma0, ma1, ma2, ma3, ma4, mb0, mb1, mb2, mb3, mb4,
                dsta_hbm, dstb_hbm, z128_hbm,
                agg0_hbm, agg1_hbm, idxa, idxb, idxt, mbufa, mbufb, zvb, acc,
                sema, semb, semia, semib):
    cid = lax.axis_index("c")
    sid = lax.axis_index("s")
    wid = sid * NC + cid
    base = wid * EPW

    pltpu.sync_copy(z128_hbm.at[pl.ds(0, G)], zvb)

    rstart = sid * 640
    rend = jnp.minimum(rstart + 640, N)

    def rowchunks(fn):
        for j in range(8):
            off = rstart + j * G
            @pl.when(off < rend)
            def _():
                fn(pl.ds(off, G))

    idxs = (idxa, idxb)
    sems = (sema, semb)
    semis = (semia, semib)

    halves = (((ma0, ma1, ma2, ma3, ma4), dsta_hbm),
              ((mb0, mb1, mb2, mb3, mb4), dstb_hbm))
    mbufs = (mbufa, mbufb)

    for g in range(5):
        rowchunks(lambda r: pltpu.sync_copy(zvb, acc.at[r]))
        plsc.subcore_barrier()

        for ms, dst_hbm in halves:
            mg = ms[g]
            pltpu.sync_copy(dst_hbm.at[pl.ds(base, TAIL)], idxt)
            pltpu.sync_copy(mg.at[pl.ds(base, TAIL)],
                            mbufa.at[pl.ds(0, TAIL)])
            pltpu.sync_copy(mbufa.at[pl.ds(0, TAIL)], acc.at[idxt],
                            add=True)

            def start(i, b):
                off = base + TAIL + i * GB
                pltpu.async_copy(dst_hbm.at[pl.ds(off, GB)], idxs[b],
                                 semis[b])
                pltpu.async_copy(mg.at[pl.ds(off, GB)], mbufs[b], sems[b])

            def commit(i, b):
                off = base + TAIL + i * GB
                pltpu.make_async_copy(dst_hbm.at[pl.ds(off, GB)], idxs[b],
                                      semis[b]).wait()
                pltpu.make_async_copy(mg.at[pl.ds(off, GB)], mbufs[b],
                                      sems[b]).wait()
                pltpu.sync_copy(mbufs[b], acc.at[idxs[b]], add=True)

            _pipeline2(NBH, start, commit)
        plsc.subcore_barrier()

        def flush(out):
            def one(r):
                pltpu.sync_copy(acc.at[r], mbufa.at[pl.ds(0, G)])
                pltpu.sync_copy(mbufa.at[pl.ds(0, G)], out.at[g, r])
            rowchunks(one)

        @pl.when(cid == 0)
        def _():
            flush(agg0_hbm)

        @pl.when(cid == 1)
        def _():
            flush(agg1_hbm)

        plsc.subcore_barrier()


def _epilogue_body(a00, a01, a02, a03, a04, a10, a11, a12, a13, a14,
                   skip_ref, res_ref, g_ref, b_ref, o_ref):
    agg = jnp.concatenate(
        [a00[...] + a10[...], a01[...] + a11[...],
         a02[...] + a12[...], a03[...] + a13[...]], axis=1)
    den8 = (a04[...] + a14[...])[:, 0:8]
    den_full = jnp.repeat(den8, OUT_CH, axis=1)
    h = agg / (den_full + 1e-16) + skip_ref[...]
    h = jnp.maximum(h, 0.0) + res_ref[...]
    mu = jnp.mean(h, axis=1, keepdims=True)
    var = jnp.mean((h - mu) ** 2, axis=1, keepdims=True)
    o_ref[...] = (h - mu) / jnp.sqrt(var + 1e-5) * g_ref[...] + b_ref[...]


def _epilogue(agg0, agg1, skip, res, g, b):
    blk = ROW_BLK
    aspec = [pl.BlockSpec((blk, 128), lambda i: (i, 0))] * 10
    return pl.pallas_call(
        _epilogue_body,
        grid=(N // blk,),
        in_specs=aspec + [
            pl.BlockSpec((blk, HID), lambda i: (i, 0)),
            pl.BlockSpec((blk, HID), lambda i: (i, 0)),
            pl.BlockSpec((1, HID), lambda i: (0, 0)),
            pl.BlockSpec((1, HID), lambda i: (0, 0)),
        ],
        out_specs=pl.BlockSpec((blk, HID), lambda i: (i, 0)),
        out_shape=jax.ShapeDtypeStruct((N, HID), jnp.float32),
    )(agg0[0], agg0[1], agg0[2], agg0[3], agg0[4],
      agg1[0], agg1[1], agg1[2], agg1[3], agg1[4],
      skip, res, g.reshape(1, HID), b.reshape(1, HID))


def _pool_body(h_ref, b_ref, sums_ref, cnt_ref):
    blk = h_ref.shape[0]
    oh = (b_ref[...] == lax.broadcasted_iota(jnp.int32, (1, NUM_GRAPHS), 1)
          ).astype(jnp.float32)
    part = lax.dot_general(oh, h_ref[...], (((0,), (0,)), ((), ())),
                           preferred_element_type=jnp.float32)
    cpart = lax.dot_general(oh, jnp.ones((blk, 128), jnp.float32),
                            (((0,), (0,)), ((), ())),
                            preferred_element_type=jnp.float32)

    @pl.when(pl.program_id(0) == 0)
    def _():
        sums_ref[...] = jnp.zeros_like(sums_ref)
        cnt_ref[...] = jnp.zeros_like(cnt_ref)

    sums_ref[...] += part
    cnt_ref[...] += cpart


def _pool(h, batch2):
    blk = ROW_BLK
    return pl.pallas_call(
        _pool_body,
        grid=(N // blk,),
        in_specs=[
            pl.BlockSpec((blk, HID), lambda i: (i, 0)),
            pl.BlockSpec((blk, 1), lambda i: (i, 0)),
        ],
        out_specs=[pl.BlockSpec((NUM_GRAPHS, HID), lambda i: (0, 0)),
                   pl.BlockSpec((NUM_GRAPHS, 128), lambda i: (0, 0))],
        out_shape=[jax.ShapeDtypeStruct((NUM_GRAPHS, HID), jnp.float32),
                   jax.ShapeDtypeStruct((NUM_GRAPHS, 128), jnp.float32)],
    )(h, batch2)


def _head_body(s_ref, c_ref, w1_ref, b1_ref, w2_ref, b2_ref, o_ref):
    cnt = jnp.maximum(c_ref[...], 1.0)
    graph = (s_ref[...].reshape(NUM_GRAPHS, 4, 128) / cnt[:, None, :]
             ).reshape(NUM_GRAPHS, HID)
    h = jnp.dot(graph, w1_ref[...], preferred_element_type=jnp.float32)
    h = jnp.maximum(h + b1_ref[...], 0.0)
    o_ref[...] = (jnp.dot(h, w2_ref[...], preferred_element_type=jnp.float32)
                  + b2_ref[...])


def _head(sums, cnt, hp):
    return pl.pallas_call(
        _head_body,
        grid=(1,),
        in_specs=[
            pl.BlockSpec((NUM_GRAPHS, HID), lambda i: (0, 0)),
            pl.BlockSpec((NUM_GRAPHS, 128), lambda i: (0, 0)),
            pl.BlockSpec((HID, OUT_CH), lambda i: (0, 0)),
            pl.BlockSpec((1, OUT_CH), lambda i: (0, 0)),
            pl.BlockSpec((OUT_CH, 1), lambda i: (0, 0)),
            pl.BlockSpec((1, 1), lambda i: (0, 0)),
        ],
        out_specs=pl.BlockSpec((NUM_GRAPHS, 1), lambda i: (0, 0)),
        out_shape=jax.ShapeDtypeStruct((NUM_GRAPHS, 1), jnp.float32),
    )(sums, cnt, hp["W1"], hp["b1"].reshape(1, OUT_CH), hp["W2"],
      hp["b2"].reshape(1, 1))


def kernel(x, params, edge_index, batch):
    src_a, src_b = edge_index[0, :E2], edge_index[0, E2:]
    dst_a, dst_b = edge_index[1, :E2], edge_index[1, E2:]
    z128 = jnp.zeros((G, 128), jnp.float32)
    cs = params["convs"]
    h = x
    res = jnp.zeros((N, HID), jnp.float32)
    for l in range(4):
        p = cs[l]
        wall = jnp.concatenate([p["Wq"], p["Wk"], p["Wv"], p["Ws"]], axis=1)
        ball = jnp.concatenate([p["bq"], p["bk"], p["bv"], p["bs"]], axis=0)
        q, k, v, skip = _proj(h, wall, ball)
        qda, ksa, vsa = _sc_gather(q, k, v, src_a, dst_a)
        ma = _edge_math(qda, ksa, vsa)
        qdb, ksb, vsb = _sc_gather(q, k, v, src_b, dst_b)
        mb = _edge_math(qdb, ksb, vsb)
        agg0, agg1 = _sc_scatter(*ma, *mb, dst_a, dst_b, z128)
        h = _epilogue(agg0, agg1, skip, res, p["ln_g"], p["ln_b"])
        res = h
    sums, cnt = _pool(h, batch.reshape(N, 1))
    return _head(sums, cnt, params["head"])
